# baseline (speedup 1.0000x reference)
import functools

import numpy as np
import jax
import jax.numpy as jnp
from jax.experimental import pallas as pl
from jax.experimental.pallas import tpu as pltpu

_PAD = 4
_DIL = 4


def _resize_matrix(out_size, in_size):
    m = np.zeros((out_size, in_size), np.float32)
    if out_size == 1:
        m[0, 0] = 1.0
        return m
    src = np.arange(out_size, dtype=np.float64) * (in_size - 1) / (out_size - 1)
    i0 = np.clip(np.floor(src).astype(np.int64), 0, in_size - 1)
    i1 = np.clip(i0 + 1, 0, in_size - 1)
    w1 = (src - i0).astype(np.float32)
    w0 = 1.0 - w1
    m[np.arange(out_size), i0] += w0
    m[np.arange(out_size), i1] += w1
    return m


def _padded_resize_matrix(out_size, in_size, pad):
    m = np.zeros((out_size + 2 * pad, in_size), np.float32)
    m[pad:pad + out_size, :] = _resize_matrix(out_size, in_size)
    return m


def _fused_kernel(C_lr, h, w, H, W,
                  x_ref, mwt_ref, mh_ref, hr_ref, dw_ref, dwb_ref,
                  wlr_ref, whr_ref, b_ref, o_ref):
    x = x_ref[0].reshape(C_lr * h, w)
    tw = jnp.dot(x, mwt_ref[...], preferred_element_type=jnp.float32)
    tw3 = tw.reshape(C_lr, h, 3 * W)

    dwv = dw_ref[...]
    gs = []
    for kh in range(3):
        g = None
        for kw in range(3):
            coef = dwv[:, kh * 3 + kw][:, None, None]
            term = coef * tw3[:, :, kw * W:(kw + 1) * W]
            g = term if g is None else g + term
        gs.append(g)
    gs.append(jnp.broadcast_to(dwb_ref[...][:, :, None], (C_lr, 1, W)))
    gbig = jnp.concatenate(gs, axis=1)

    mb = jnp.broadcast_to(mh_ref[...], (C_lr, H, 3 * h + 1))
    acc = jax.lax.dot_general(
        mb, gbig, (((2,), (1,)), ((0,), (0,))),
        preferred_element_type=jnp.float32)

    y = jnp.maximum(acc.reshape(C_lr, H * W), 0.0)
    o = jnp.dot(wlr_ref[...], y, preferred_element_type=jnp.float32)
    o = o + jnp.dot(whr_ref[...], hr_ref[0], preferred_element_type=jnp.float32)
    o_ref[0] = jnp.maximum(o + b_ref[...], 0.0)


def kernel(lowres, highres, dw_w_eff, dw_bias_f, w_lr_eff, w_hr_eff, b_sum):
    N, C_lr, h, w = lowres.shape
    _, C_hr, H, W = highres.shape
    C_out = w_lr_eff.shape[0]

    mw_pad = _padded_resize_matrix(W, w, _PAD)
    mh_pad = _padded_resize_matrix(H, h, _PAD)
    mwt = np.concatenate(
        [mw_pad[kw * _DIL: kw * _DIL + W, :].T for kw in range(3)], axis=1)
    mbig = np.concatenate(
        [mh_pad[kh * _DIL: kh * _DIL + H, :] for kh in range(3)]
        + [np.ones((H, 1), np.float32)], axis=1)

    mwt = jnp.asarray(mwt)
    mbig = jnp.asarray(mbig)
    dwb = dw_bias_f.reshape(C_lr, 1)
    S = H * W

    kern = functools.partial(_fused_kernel, C_lr, h, w, H, W)
    out = pl.pallas_call(
        kern,
        out_shape=jax.ShapeDtypeStruct((N, C_out, S), jnp.float32),
        grid=(N,),
        in_specs=[
            pl.BlockSpec((1, C_lr, h, w), lambda n: (n, 0, 0, 0)),
            pl.BlockSpec((w, 3 * W), lambda n: (0, 0)),
            pl.BlockSpec((H, 3 * h + 1), lambda n: (0, 0)),
            pl.BlockSpec((1, C_hr, S), lambda n: (n, 0, 0)),
            pl.BlockSpec((C_lr, 9), lambda n: (0, 0)),
            pl.BlockSpec((C_lr, 1), lambda n: (0, 0)),
            pl.BlockSpec((C_out, C_lr), lambda n: (0, 0)),
            pl.BlockSpec((C_out, C_hr), lambda n: (0, 0)),
            pl.BlockSpec((C_out, 1), lambda n: (0, 0)),
        ],
        out_specs=pl.BlockSpec((1, C_out, S), lambda n: (n, 0, 0)),
        compiler_params=pltpu.CompilerParams(
            dimension_semantics=("parallel",),
            vmem_limit_bytes=60 * 1024 * 1024,
        ),
    )(lowres, mwt, mbig, highres.reshape(N, C_hr, S),
      dw_w_eff, dwb, w_lr_eff, w_hr_eff, b_sum)
    return out.reshape(N, C_out, H, W)


# speedup vs baseline: 1.3431x; 1.3431x over previous
import functools

import numpy as np
import jax
import jax.numpy as jnp
from jax.experimental import pallas as pl
from jax.experimental.pallas import tpu as pltpu

_PAD = 4
_DIL = 4


def _resize_matrix(out_size, in_size):
    m = np.zeros((out_size, in_size), np.float32)
    if out_size == 1:
        m[0, 0] = 1.0
        return m
    src = np.arange(out_size, dtype=np.float64) * (in_size - 1) / (out_size - 1)
    i0 = np.clip(np.floor(src).astype(np.int64), 0, in_size - 1)
    i1 = np.clip(i0 + 1, 0, in_size - 1)
    w1 = (src - i0).astype(np.float32)
    w0 = 1.0 - w1
    m[np.arange(out_size), i0] += w0
    m[np.arange(out_size), i1] += w1
    return m


def _padded_resize_matrix(out_size, in_size, pad):
    m = np.zeros((out_size + 2 * pad, in_size), np.float32)
    m[pad:pad + out_size, :] = _resize_matrix(out_size, in_size)
    return m


def _fused_kernel(C_lr, h, w, H, W,
                  x_ref, mwt_ref, mh_ref, hr_ref, dw_ref, dwb_ref,
                  wlr_ref, whr_ref, b_ref, o_ref):
    x = x_ref[0].reshape(C_lr * h, w)
    tw = jnp.dot(x, mwt_ref[...], preferred_element_type=jnp.float32)
    tw3 = tw.reshape(C_lr, h, 3 * W)

    dwv = dw_ref[...]
    gs = []
    for kh in range(3):
        g = None
        for kw in range(3):
            coef = dwv[:, kh * 3 + kw][:, None, None]
            term = coef * tw3[:, :, kw * W:(kw + 1) * W]
            g = term if g is None else g + term
        gs.append(g)
    gs.append(jnp.broadcast_to(dwb_ref[...][:, :, None], (C_lr, 1, W)))
    gbig = jnp.concatenate(gs, axis=1)

    mb = jnp.broadcast_to(mh_ref[...], (C_lr, H, 3 * h + 1))
    acc = jax.lax.dot_general(
        mb, gbig, (((2,), (1,)), ((0,), (0,))),
        preferred_element_type=jnp.float32)

    y = jnp.maximum(acc.reshape(C_lr, H * W), 0.0)
    o = jnp.dot(wlr_ref[...], y, preferred_element_type=jnp.float32)
    o = o + jnp.dot(whr_ref[...], hr_ref[0], preferred_element_type=jnp.float32)
    o = jnp.maximum(o + b_ref[...], 0.0)
    o_ref[0] = o.reshape(o_ref.shape[1], H, W)


def kernel(lowres, highres, dw_w_eff, dw_bias_f, w_lr_eff, w_hr_eff, b_sum):
    N, C_lr, h, w = lowres.shape
    _, C_hr, H, W = highres.shape
    C_out = w_lr_eff.shape[0]

    mw_pad = _padded_resize_matrix(W, w, _PAD)
    mh_pad = _padded_resize_matrix(H, h, _PAD)
    mwt = np.concatenate(
        [mw_pad[kw * _DIL: kw * _DIL + W, :].T for kw in range(3)], axis=1)
    mbig = np.concatenate(
        [mh_pad[kh * _DIL: kh * _DIL + H, :] for kh in range(3)]
        + [np.ones((H, 1), np.float32)], axis=1)

    mwt = jnp.asarray(mwt)
    mbig = jnp.asarray(mbig)
    dwb = dw_bias_f.reshape(C_lr, 1)
    S = H * W

    kern = functools.partial(_fused_kernel, C_lr, h, w, H, W)
    return pl.pallas_call(
        kern,
        out_shape=jax.ShapeDtypeStruct((N, C_out, H, W), jnp.float32),
        grid=(N,),
        in_specs=[
            pl.BlockSpec((1, C_lr, h, w), lambda n: (n, 0, 0, 0)),
            pl.BlockSpec((w, 3 * W), lambda n: (0, 0)),
            pl.BlockSpec((H, 3 * h + 1), lambda n: (0, 0)),
            pl.BlockSpec((1, C_hr, S), lambda n: (n, 0, 0)),
            pl.BlockSpec((C_lr, 9), lambda n: (0, 0)),
            pl.BlockSpec((C_lr, 1), lambda n: (0, 0)),
            pl.BlockSpec((C_out, C_lr), lambda n: (0, 0)),
            pl.BlockSpec((C_out, C_hr), lambda n: (0, 0)),
            pl.BlockSpec((C_out, 1), lambda n: (0, 0)),
        ],
        out_specs=pl.BlockSpec((1, C_out, H, W), lambda n: (n, 0, 0, 0)),
        compiler_params=pltpu.CompilerParams(
            dimension_semantics=("parallel",),
            vmem_limit_bytes=60 * 1024 * 1024,
        ),
    )(lowres, mwt, mbig, highres.reshape(N, C_hr, S),
      dw_w_eff, dwb, w_lr_eff, w_hr_eff, b_sum)


# speedup vs baseline: 1.5124x; 1.1261x over previous
import functools

import numpy as np
import jax
import jax.numpy as jnp
from jax.experimental import pallas as pl
from jax.experimental.pallas import tpu as pltpu

_PAD = 4
_DIL = 4
_T = 4


def _resize_matrix(out_size, in_size):
    m = np.zeros((out_size, in_size), np.float32)
    if out_size == 1:
        m[0, 0] = 1.0
        return m
    src = np.arange(out_size, dtype=np.float64) * (in_size - 1) / (out_size - 1)
    i0 = np.clip(np.floor(src).astype(np.int64), 0, in_size - 1)
    i1 = np.clip(i0 + 1, 0, in_size - 1)
    w1 = (src - i0).astype(np.float32)
    w0 = 1.0 - w1
    m[np.arange(out_size), i0] += w0
    m[np.arange(out_size), i1] += w1
    return m


def _padded_resize_matrix(out_size, in_size, pad):
    m = np.zeros((out_size + 2 * pad, in_size), np.float32)
    m[pad:pad + out_size, :] = _resize_matrix(out_size, in_size)
    return m


def _fused_kernel(C_lr, h, w, H, W, TH,
                  x_ref, mwt_ref, mh_ref, hr_ref, dw_ref, dwb_ref,
                  wlr_ref, whr_ref, b_ref, o_ref, gbig_ref):
    t = pl.program_id(1)

    @pl.when(t == 0)
    def _prep():
        x = x_ref[0].reshape(C_lr * h, w)
        tw = jnp.dot(x, mwt_ref[...], preferred_element_type=jnp.float32)
        tw3 = tw.reshape(C_lr, h, 3 * W)
        dwv = dw_ref[...]
        gs = []
        for kh in range(3):
            g = None
            for kw in range(3):
                coef = dwv[:, kh * 3 + kw][:, None, None]
                term = coef * tw3[:, :, kw * W:(kw + 1) * W]
                g = term if g is None else g + term
            gs.append(g)
        gs.append(jnp.broadcast_to(dwb_ref[...][:, :, None], (C_lr, 1, W)))
        gbig_ref[...] = jnp.concatenate(gs, axis=1)

    mb = jnp.broadcast_to(mh_ref[0], (C_lr, TH, 3 * h + 1))
    acc = jax.lax.dot_general(
        mb, gbig_ref[...], (((2,), (1,)), ((0,), (0,))),
        preferred_element_type=jnp.float32)

    y = jnp.maximum(acc.reshape(C_lr, TH * W), 0.0)
    hr = hr_ref[0].reshape(hr_ref.shape[1], TH * W)
    o = jnp.dot(wlr_ref[...], y, preferred_element_type=jnp.float32)
    o = o + jnp.dot(whr_ref[...], hr, preferred_element_type=jnp.float32)
    o = jnp.maximum(o + b_ref[...], 0.0)
    o_ref[0] = o.reshape(o_ref.shape[1], TH, W)


def kernel(lowres, highres, dw_w_eff, dw_bias_f, w_lr_eff, w_hr_eff, b_sum):
    N, C_lr, h, w = lowres.shape
    _, C_hr, H, W = highres.shape
    C_out = w_lr_eff.shape[0]
    TH = H // _T
    K3 = 3 * h + 1

    mw_pad = _padded_resize_matrix(W, w, _PAD)
    mh_pad = _padded_resize_matrix(H, h, _PAD)
    mwt = np.concatenate(
        [mw_pad[kw * _DIL: kw * _DIL + W, :].T for kw in range(3)], axis=1)
    mbig = np.concatenate(
        [mh_pad[kh * _DIL: kh * _DIL + H, :] for kh in range(3)]
        + [np.ones((H, 1), np.float32)], axis=1).reshape(_T, TH, K3)

    mwt = jnp.asarray(mwt)
    mbig = jnp.asarray(mbig)
    dwb = dw_bias_f.reshape(C_lr, 1)

    kern = functools.partial(_fused_kernel, C_lr, h, w, H, W, TH)
    return pl.pallas_call(
        kern,
        out_shape=jax.ShapeDtypeStruct((N, C_out, H, W), jnp.float32),
        grid=(N, _T),
        in_specs=[
            pl.BlockSpec((1, C_lr, h, w), lambda n, t: (n, 0, 0, 0)),
            pl.BlockSpec((w, 3 * W), lambda n, t: (0, 0)),
            pl.BlockSpec((1, TH, K3), lambda n, t: (t, 0, 0)),
            pl.BlockSpec((1, C_hr, TH, W), lambda n, t: (n, 0, t, 0)),
            pl.BlockSpec((C_lr, 9), lambda n, t: (0, 0)),
            pl.BlockSpec((C_lr, 1), lambda n, t: (0, 0)),
            pl.BlockSpec((C_out, C_lr), lambda n, t: (0, 0)),
            pl.BlockSpec((C_out, C_hr), lambda n, t: (0, 0)),
            pl.BlockSpec((C_out, 1), lambda n, t: (0, 0)),
        ],
        out_specs=pl.BlockSpec((1, C_out, TH, W), lambda n, t: (n, 0, t, 0)),
        scratch_shapes=[pltpu.VMEM((C_lr, K3, W), jnp.float32)],
        compiler_params=pltpu.CompilerParams(
            dimension_semantics=("parallel", "arbitrary"),
            vmem_limit_bytes=60 * 1024 * 1024,
        ),
    )(lowres, mwt, mbig, highres, dw_w_eff, dwb, w_lr_eff, w_hr_eff, b_sum)


# speedup vs baseline: 1.7528x; 1.1590x over previous
import functools

import numpy as np
import jax
import jax.numpy as jnp
from jax.experimental import pallas as pl
from jax.experimental.pallas import tpu as pltpu

_PAD = 4
_DIL = 4


def _resize_matrix(out_size, in_size):
    m = np.zeros((out_size, in_size), np.float32)
    if out_size == 1:
        m[0, 0] = 1.0
        return m
    src = np.arange(out_size, dtype=np.float64) * (in_size - 1) / (out_size - 1)
    i0 = np.clip(np.floor(src).astype(np.int64), 0, in_size - 1)
    i1 = np.clip(i0 + 1, 0, in_size - 1)
    w1 = (src - i0).astype(np.float32)
    w0 = 1.0 - w1
    m[np.arange(out_size), i0] += w0
    m[np.arange(out_size), i1] += w1
    return m


def _padded_resize_matrix(out_size, in_size, pad):
    m = np.zeros((out_size + 2 * pad, in_size), np.float32)
    m[pad:pad + out_size, :] = _resize_matrix(out_size, in_size)
    return m


def _fused_kernel(C_lr, h, w, H, W,
                  x_ref, mwt_ref, mh_ref, hr_ref, dw_ref, dwb_ref,
                  wlr_ref, whr_ref, b_ref, o_ref, gbig_ref):
    x = x_ref[0].reshape(C_lr * h, w)
    tw = jnp.dot(x, mwt_ref[...], preferred_element_type=jnp.float32)
    tw3 = tw.reshape(C_lr, h, 3 * W)

    dwv = dw_ref[...]
    for kh in range(3):
        g = None
        for kw in range(3):
            coef = dwv[:, kh * 3 + kw][:, None, None]
            term = coef * tw3[:, :, kw * W:(kw + 1) * W]
            g = term if g is None else g + term
        gbig_ref[:, kh * h:(kh + 1) * h, :] = g
    gbig_ref[:, 3 * h:, :] = jnp.broadcast_to(
        dwb_ref[...][:, :, None], (C_lr, 1, W))

    mb = jnp.broadcast_to(mh_ref[...], (C_lr, H, 3 * h + 1))
    acc = jax.lax.dot_general(
        mb, gbig_ref[...], (((2,), (1,)), ((0,), (0,))),
        preferred_element_type=jnp.float32)

    y = jnp.maximum(acc.reshape(C_lr, H * W), 0.0)
    hr = hr_ref[0].reshape(hr_ref.shape[1], H * W)
    o = jnp.dot(wlr_ref[...], y, preferred_element_type=jnp.float32)
    o = o + jnp.dot(whr_ref[...], hr, preferred_element_type=jnp.float32)
    o = jnp.maximum(o + b_ref[...], 0.0)
    o_ref[0] = o.reshape(o_ref.shape[1], H, W)


def kernel(lowres, highres, dw_w_eff, dw_bias_f, w_lr_eff, w_hr_eff, b_sum):
    N, C_lr, h, w = lowres.shape
    _, C_hr, H, W = highres.shape
    C_out = w_lr_eff.shape[0]
    K3 = 3 * h + 1

    mw_pad = _padded_resize_matrix(W, w, _PAD)
    mh_pad = _padded_resize_matrix(H, h, _PAD)
    mwt = np.concatenate(
        [mw_pad[kw * _DIL: kw * _DIL + W, :].T for kw in range(3)], axis=1)
    mbig = np.concatenate(
        [mh_pad[kh * _DIL: kh * _DIL + H, :] for kh in range(3)]
        + [np.ones((H, 1), np.float32)], axis=1)

    mwt = jnp.asarray(mwt)
    mbig = jnp.asarray(mbig)
    dwb = dw_bias_f.reshape(C_lr, 1)

    kern = functools.partial(_fused_kernel, C_lr, h, w, H, W)
    return pl.pallas_call(
        kern,
        out_shape=jax.ShapeDtypeStruct((N, C_out, H, W), jnp.float32),
        grid=(N,),
        in_specs=[
            pl.BlockSpec((1, C_lr, h, w), lambda n: (n, 0, 0, 0)),
            pl.BlockSpec((w, 3 * W), lambda n: (0, 0)),
            pl.BlockSpec((H, K3), lambda n: (0, 0)),
            pl.BlockSpec((1, C_hr, H, W), lambda n: (n, 0, 0, 0)),
            pl.BlockSpec((C_lr, 9), lambda n: (0, 0)),
            pl.BlockSpec((C_lr, 1), lambda n: (0, 0)),
            pl.BlockSpec((C_out, C_lr), lambda n: (0, 0)),
            pl.BlockSpec((C_out, C_hr), lambda n: (0, 0)),
            pl.BlockSpec((C_out, 1), lambda n: (0, 0)),
        ],
        out_specs=pl.BlockSpec((1, C_out, H, W), lambda n: (n, 0, 0, 0)),
        scratch_shapes=[pltpu.VMEM((C_lr, K3, W), jnp.float32)],
        compiler_params=pltpu.CompilerParams(
            dimension_semantics=("parallel",),
            vmem_limit_bytes=60 * 1024 * 1024,
        ),
    )(lowres, mwt, mbig, highres, dw_w_eff, dwb, w_lr_eff, w_hr_eff, b_sum)


# speedup vs baseline: 1.8686x; 1.0661x over previous
import functools

import numpy as np
import jax
import jax.numpy as jnp
from jax.experimental import pallas as pl
from jax.experimental.pallas import tpu as pltpu

_PAD = 4
_DIL = 4


def _resize_matrix(out_size, in_size):
    m = np.zeros((out_size, in_size), np.float32)
    if out_size == 1:
        m[0, 0] = 1.0
        return m
    src = np.arange(out_size, dtype=np.float64) * (in_size - 1) / (out_size - 1)
    i0 = np.clip(np.floor(src).astype(np.int64), 0, in_size - 1)
    i1 = np.clip(i0 + 1, 0, in_size - 1)
    w1 = (src - i0).astype(np.float32)
    w0 = 1.0 - w1
    m[np.arange(out_size), i0] += w0
    m[np.arange(out_size), i1] += w1
    return m


def _padded_resize_matrix(out_size, in_size, pad):
    m = np.zeros((out_size + 2 * pad, in_size), np.float32)
    m[pad:pad + out_size, :] = _resize_matrix(out_size, in_size)
    return m


def _fused_kernel(C_lr, h, w, H, W,
                  x_ref, mwt_ref, mh_ref, hr_ref, dw_ref, dwb_ref,
                  wlr_ref, whr_ref, b_ref, o_ref, gbig_ref):
    x = x_ref[0].reshape(C_lr * h, w)
    tw = jnp.dot(x, mwt_ref[...], preferred_element_type=jnp.float32)
    tw3 = tw.reshape(C_lr, h, 3 * W)

    dwv = dw_ref[...]
    for kh in range(3):
        g = None
        for kw in range(3):
            coef = dwv[:, kh * 3 + kw][:, None, None]
            term = coef * tw3[:, :, kw * W:(kw + 1) * W]
            g = term if g is None else g + term
        gbig_ref[:, kh * h:(kh + 1) * h, :] = g
    gbig_ref[:, 3 * h:, :] = jnp.broadcast_to(
        dwb_ref[...][:, :, None], (C_lr, 1, W))

    mb = jnp.broadcast_to(mh_ref[...], (C_lr, H, 3 * h + 1))
    acc = jax.lax.dot_general(
        mb, gbig_ref[...], (((2,), (1,)), ((0,), (0,))),
        preferred_element_type=jnp.float32)

    y = jnp.maximum(acc, 0.0).astype(jnp.bfloat16).reshape(C_lr, H * W)
    hr = hr_ref[0].astype(jnp.bfloat16).reshape(hr_ref.shape[1], H * W)
    o = jnp.dot(wlr_ref[...], y, preferred_element_type=jnp.float32)
    o = o + jnp.dot(whr_ref[...], hr, preferred_element_type=jnp.float32)
    o = jnp.maximum(o + b_ref[...], 0.0)
    o_ref[0] = o.reshape(o_ref.shape[1], H, W)


def kernel(lowres, highres, dw_w_eff, dw_bias_f, w_lr_eff, w_hr_eff, b_sum):
    N, C_lr, h, w = lowres.shape
    _, C_hr, H, W = highres.shape
    C_out = w_lr_eff.shape[0]
    K3 = 3 * h + 1

    mw_pad = _padded_resize_matrix(W, w, _PAD)
    mh_pad = _padded_resize_matrix(H, h, _PAD)
    mwt = np.concatenate(
        [mw_pad[kw * _DIL: kw * _DIL + W, :].T for kw in range(3)], axis=1)
    mbig = np.concatenate(
        [mh_pad[kh * _DIL: kh * _DIL + H, :] for kh in range(3)]
        + [np.ones((H, 1), np.float32)], axis=1)

    mwt = jnp.asarray(mwt)
    mbig = jnp.asarray(mbig)
    dwb = dw_bias_f.reshape(C_lr, 1)

    kern = functools.partial(_fused_kernel, C_lr, h, w, H, W)
    return pl.pallas_call(
        kern,
        out_shape=jax.ShapeDtypeStruct((N, C_out, H, W), jnp.float32),
        grid=(N,),
        in_specs=[
            pl.BlockSpec((1, C_lr, h, w), lambda n: (n, 0, 0, 0)),
            pl.BlockSpec((w, 3 * W), lambda n: (0, 0)),
            pl.BlockSpec((H, K3), lambda n: (0, 0)),
            pl.BlockSpec((1, C_hr, H, W), lambda n: (n, 0, 0, 0)),
            pl.BlockSpec((C_lr, 9), lambda n: (0, 0)),
            pl.BlockSpec((C_lr, 1), lambda n: (0, 0)),
            pl.BlockSpec((C_out, C_lr), lambda n: (0, 0)),
            pl.BlockSpec((C_out, C_hr), lambda n: (0, 0)),
            pl.BlockSpec((C_out, 1), lambda n: (0, 0)),
        ],
        out_specs=pl.BlockSpec((1, C_out, H, W), lambda n: (n, 0, 0, 0)),
        scratch_shapes=[pltpu.VMEM((C_lr, K3, W), jnp.float32)],
        compiler_params=pltpu.CompilerParams(
            dimension_semantics=("parallel",),
            vmem_limit_bytes=60 * 1024 * 1024,
        ),
    )(lowres, mwt, mbig, highres, dw_w_eff, dwb,
      w_lr_eff.astype(jnp.bfloat16), w_hr_eff.astype(jnp.bfloat16), b_sum)


# speedup vs baseline: 1.9538x; 1.0456x over previous
import functools

import numpy as np
import jax
import jax.numpy as jnp
from jax.experimental import pallas as pl
from jax.experimental.pallas import tpu as pltpu

_PAD = 4
_DIL = 4


def _resize_matrix(out_size, in_size):
    m = np.zeros((out_size, in_size), np.float32)
    if out_size == 1:
        m[0, 0] = 1.0
        return m
    src = np.arange(out_size, dtype=np.float64) * (in_size - 1) / (out_size - 1)
    i0 = np.clip(np.floor(src).astype(np.int64), 0, in_size - 1)
    i1 = np.clip(i0 + 1, 0, in_size - 1)
    w1 = (src - i0).astype(np.float32)
    w0 = 1.0 - w1
    m[np.arange(out_size), i0] += w0
    m[np.arange(out_size), i1] += w1
    return m


def _padded_resize_matrix(out_size, in_size, pad):
    m = np.zeros((out_size + 2 * pad, in_size), np.float32)
    m[pad:pad + out_size, :] = _resize_matrix(out_size, in_size)
    return m


def _fused_kernel(C_lr, h, w, H, W,
                  x_ref, mwt_ref, mh_ref, hr_ref, dw_ref, dwb_ref,
                  wlr_ref, whr_ref, b_ref, o_ref, gbig_ref):
    x = x_ref[0].reshape(C_lr * h, w)
    tw = jnp.dot(x, mwt_ref[...], preferred_element_type=jnp.float32)
    tw3 = tw.astype(jnp.bfloat16).reshape(C_lr, h, 3 * W)

    dwv = dw_ref[...]
    for kh in range(3):
        g = None
        for kw in range(3):
            coef = dwv[:, kh * 3 + kw][:, None, None]
            term = coef * tw3[:, :, kw * W:(kw + 1) * W]
            g = term if g is None else g + term
        gbig_ref[:, kh * h:(kh + 1) * h, :] = g
    gbig_ref[:, 3 * h:, :] = jnp.broadcast_to(
        dwb_ref[...][:, :, None], (C_lr, 1, W))

    mb = jnp.broadcast_to(mh_ref[...], (C_lr, H, 3 * h + 1))
    acc = jax.lax.dot_general(
        mb, gbig_ref[...], (((2,), (1,)), ((0,), (0,))),
        preferred_element_type=jnp.float32)

    y = jnp.maximum(acc, 0.0).astype(jnp.bfloat16).reshape(C_lr, H * W)
    hr = hr_ref[0].astype(jnp.bfloat16).reshape(hr_ref.shape[1], H * W)
    o = jnp.dot(wlr_ref[...], y, preferred_element_type=jnp.float32)
    o = o + jnp.dot(whr_ref[...], hr, preferred_element_type=jnp.float32)
    o = jnp.maximum(o + b_ref[...], 0.0)
    o_ref[0] = o.reshape(o_ref.shape[1], H, W)


def kernel(lowres, highres, dw_w_eff, dw_bias_f, w_lr_eff, w_hr_eff, b_sum):
    N, C_lr, h, w = lowres.shape
    _, C_hr, H, W = highres.shape
    C_out = w_lr_eff.shape[0]
    K3 = 3 * h + 1

    mw_pad = _padded_resize_matrix(W, w, _PAD)
    mh_pad = _padded_resize_matrix(H, h, _PAD)
    mwt = np.concatenate(
        [mw_pad[kw * _DIL: kw * _DIL + W, :].T for kw in range(3)], axis=1)
    mbig = np.concatenate(
        [mh_pad[kh * _DIL: kh * _DIL + H, :] for kh in range(3)]
        + [np.ones((H, 1), np.float32)], axis=1)

    mwt = jnp.asarray(mwt)
    mbig = jnp.asarray(mbig)
    dwb = dw_bias_f.reshape(C_lr, 1)

    kern = functools.partial(_fused_kernel, C_lr, h, w, H, W)
    return pl.pallas_call(
        kern,
        out_shape=jax.ShapeDtypeStruct((N, C_out, H, W), jnp.float32),
        grid=(N,),
        in_specs=[
            pl.BlockSpec((1, C_lr, h, w), lambda n: (n, 0, 0, 0)),
            pl.BlockSpec((w, 3 * W), lambda n: (0, 0)),
            pl.BlockSpec((H, K3), lambda n: (0, 0)),
            pl.BlockSpec((1, C_hr, H, W), lambda n: (n, 0, 0, 0)),
            pl.BlockSpec((C_lr, 9), lambda n: (0, 0)),
            pl.BlockSpec((C_lr, 1), lambda n: (0, 0)),
            pl.BlockSpec((C_out, C_lr), lambda n: (0, 0)),
            pl.BlockSpec((C_out, C_hr), lambda n: (0, 0)),
            pl.BlockSpec((C_out, 1), lambda n: (0, 0)),
        ],
        out_specs=pl.BlockSpec((1, C_out, H, W), lambda n: (n, 0, 0, 0)),
        scratch_shapes=[pltpu.VMEM((C_lr, K3, W), jnp.bfloat16)],
        compiler_params=pltpu.CompilerParams(
            dimension_semantics=("parallel",),
            vmem_limit_bytes=60 * 1024 * 1024,
        ),
    )(lowres, mwt, mbig.astype(jnp.bfloat16), highres,
      dw_w_eff.astype(jnp.bfloat16), dwb.astype(jnp.bfloat16),
      w_lr_eff.astype(jnp.bfloat16), w_hr_eff.astype(jnp.bfloat16), b_sum)


# speedup vs baseline: 1.9581x; 1.0022x over previous
import functools

import numpy as np
import jax
import jax.numpy as jnp
from jax.experimental import pallas as pl
from jax.experimental.pallas import tpu as pltpu

_PAD = 4
_DIL = 4


def _resize_matrix(out_size, in_size):
    m = np.zeros((out_size, in_size), np.float32)
    if out_size == 1:
        m[0, 0] = 1.0
        return m
    src = np.arange(out_size, dtype=np.float64) * (in_size - 1) / (out_size - 1)
    i0 = np.clip(np.floor(src).astype(np.int64), 0, in_size - 1)
    i1 = np.clip(i0 + 1, 0, in_size - 1)
    w1 = (src - i0).astype(np.float32)
    w0 = 1.0 - w1
    m[np.arange(out_size), i0] += w0
    m[np.arange(out_size), i1] += w1
    return m


def _padded_resize_matrix(out_size, in_size, pad):
    m = np.zeros((out_size + 2 * pad, in_size), np.float32)
    m[pad:pad + out_size, :] = _resize_matrix(out_size, in_size)
    return m


def _fused_kernel(C_lr, h, w, H, W,
                  x_ref, mwt_ref, mh_ref, hr_ref, dw_ref, dwb_ref,
                  wlr_ref, whr_ref, b_ref, o_ref, gbig_ref):
    x = x_ref[0].reshape(C_lr * h, w)
    tw = jnp.dot(x, mwt_ref[...], preferred_element_type=jnp.float32)
    tw3 = tw.astype(jnp.bfloat16).reshape(C_lr, h, 3 * W)

    dwv = dw_ref[...]
    for kh in range(3):
        g = None
        for kw in range(3):
            coef = dwv[:, kh * 3 + kw][:, None, None]
            term = coef * tw3[:, :, kw * W:(kw + 1) * W]
            g = term if g is None else g + term
        gbig_ref[:, kh * h:(kh + 1) * h, :] = g
    gbig_ref[:, 3 * h:, :] = jnp.broadcast_to(
        dwb_ref[...][:, :, None], (C_lr, 1, W))

    mb = jnp.broadcast_to(mh_ref[...], (C_lr, H, 3 * h + 1))
    acc = jax.lax.dot_general(
        mb, gbig_ref[...], (((2,), (1,)), ((0,), (0,))),
        preferred_element_type=jnp.float32)

    y = jnp.maximum(acc, 0.0).astype(jnp.bfloat16).reshape(C_lr, H * W)
    hr = hr_ref[0].astype(jnp.bfloat16).reshape(hr_ref.shape[1], H * W)
    o = jnp.dot(wlr_ref[...], y, preferred_element_type=jnp.float32)
    o = o + jnp.dot(whr_ref[...], hr, preferred_element_type=jnp.float32)
    o = jnp.maximum(o + b_ref[...], 0.0)
    o_ref[0] = o.reshape(o_ref.shape[1], H, W)


def kernel(lowres, highres, dw_w_eff, dw_bias_f, w_lr_eff, w_hr_eff, b_sum):
    N, C_lr, h, w = lowres.shape
    _, C_hr, H, W = highres.shape
    C_out = w_lr_eff.shape[0]
    K3 = 3 * h + 1

    mw_pad = _padded_resize_matrix(W, w, _PAD)
    mh_pad = _padded_resize_matrix(H, h, _PAD)
    mwt = np.concatenate(
        [mw_pad[kw * _DIL: kw * _DIL + W, :].T for kw in range(3)], axis=1)
    mbig = np.concatenate(
        [mh_pad[kh * _DIL: kh * _DIL + H, :] for kh in range(3)]
        + [np.ones((H, 1), np.float32)], axis=1)

    mwt = jnp.asarray(mwt)
    dwb = dw_bias_f.reshape(C_lr, 1)

    kern = functools.partial(_fused_kernel, C_lr, h, w, H, W)
    return pl.pallas_call(
        kern,
        out_shape=jax.ShapeDtypeStruct((N, C_out, H, W), jnp.float32),
        grid=(N,),
        in_specs=[
            pl.BlockSpec((1, C_lr, h, w), lambda n: (n, 0, 0, 0)),
            pl.BlockSpec((w, 3 * W), lambda n: (0, 0)),
            pl.BlockSpec((H, K3), lambda n: (0, 0)),
            pl.BlockSpec((1, C_hr, H, W), lambda n: (n, 0, 0, 0)),
            pl.BlockSpec((C_lr, 9), lambda n: (0, 0)),
            pl.BlockSpec((C_lr, 1), lambda n: (0, 0)),
            pl.BlockSpec((C_out, C_lr), lambda n: (0, 0)),
            pl.BlockSpec((C_out, C_hr), lambda n: (0, 0)),
            pl.BlockSpec((C_out, 1), lambda n: (0, 0)),
        ],
        out_specs=pl.BlockSpec((1, C_out, H, W), lambda n: (n, 0, 0, 0)),
        scratch_shapes=[pltpu.VMEM((C_lr, K3, W), jnp.bfloat16)],
        compiler_params=pltpu.CompilerParams(
            dimension_semantics=("parallel",),
            vmem_limit_bytes=60 * 1024 * 1024,
        ),
    )(lowres, mwt, jnp.asarray(mbig).astype(jnp.bfloat16), highres,
      dw_w_eff.astype(jnp.bfloat16), dwb.astype(jnp.bfloat16),
      w_lr_eff.astype(jnp.bfloat16), w_hr_eff.astype(jnp.bfloat16), b_sum)
